# baseline (speedup 1.0000x reference)
import jax
import jax.numpy as jnp
import numpy as np
from jax.experimental import pallas as pl
from jax.experimental.pallas import tpu as pltpu

A_MAX = 7.25
S_MAX = 9.25
B = 128
J = 22
F = 6600
NBUF = 4
NG = B // NBUF


def _field_xy():
    x = np.linspace(0.5, 119.5, 120, dtype=np.float32)
    y = np.linspace(-0.5, 53.5, 55, dtype=np.float32)
    y[0] = -0.2
    yy, xx = np.meshgrid(y, x, indexing="ij")
    return xx.reshape(1, F), yy.reshape(1, F)


def _compute(x, y, vx, vy, fx, fy):
    dx = fx - x
    dy = fy - y
    d2 = dx * dx + dy * dy
    r = jax.lax.rsqrt(d2)
    d = d2 * r
    s0 = jnp.clip((dx * vx + dy * vy) * r, -S_MAX, S_MAX)
    s02 = s0 * s0
    dlt = (S_MAX * S_MAX / (2.0 * A_MAX)) - s02 * (0.5 / A_MAX)
    qq = s02 + (2.0 * A_MAX) * d
    sq = qq * jax.lax.rsqrt(qq)
    us = s0 * (1.0 / A_MAX)
    t2 = sq * (1.0 / A_MAX) - us
    t1 = (S_MAX / A_MAX) - us
    tl = jnp.where(dlt > d, t2, t1)
    dd = jnp.maximum(d - jnp.maximum(dlt, 0.0), 0.0)
    return tl + dd * (1.0 / S_MAX)


def _body(x_ref, y_ref, vx_ref, vy_ref, fx_ref, fy_ref, out_hbm,
          buf, sem0, sem1, sem2, sem3):
    g = pl.program_id(0)
    sems = (sem0, sem1, sem2, sem3)
    fx = fx_ref[...]
    fy = fy_ref[...]
    for s in range(NBUF):
        b = g * NBUF + s

        @pl.when(g > 0)
        def _wait(s=s):
            pltpu.make_async_copy(
                buf.at[s], out_hbm.at[pl.ds(0, 1)], sems[s]).wait()

        t = _compute(x_ref[s], y_ref[s], vx_ref[s], vy_ref[s], fx, fy)
        buf[s] = t.T[None]
        pltpu.async_copy(buf.at[s], out_hbm.at[pl.ds(b, 1)], sems[s])

    @pl.when(g == NG - 1)
    def _drain():
        for s in range(NBUF):
            pltpu.make_async_copy(
                buf.at[s], out_hbm.at[pl.ds(0, 1)], sems[s]).wait()


@jax.jit
def _run(xp, yp, vxp, vyp, fx, fy):
    return pl.pallas_call(
        _body,
        grid=(NG,),
        in_specs=[
            pl.BlockSpec((NBUF, J, 1), lambda g: (g, 0, 0)),
            pl.BlockSpec((NBUF, J, 1), lambda g: (g, 0, 0)),
            pl.BlockSpec((NBUF, J, 1), lambda g: (g, 0, 0)),
            pl.BlockSpec((NBUF, J, 1), lambda g: (g, 0, 0)),
            pl.BlockSpec((1, F), lambda g: (0, 0)),
            pl.BlockSpec((1, F), lambda g: (0, 0)),
        ],
        out_specs=pl.BlockSpec(memory_space=pltpu.MemorySpace.HBM),
        out_shape=jax.ShapeDtypeStruct((B, F, J), jnp.float32),
        scratch_shapes=[
            pltpu.VMEM((NBUF, 1, F, J), jnp.float32),
            pltpu.SemaphoreType.DMA,
            pltpu.SemaphoreType.DMA,
            pltpu.SemaphoreType.DMA,
            pltpu.SemaphoreType.DMA,
        ],
    )(xp, yp, vxp, vyp, fx, fy)


def kernel(frame):
    xp = frame[:, :, 1:2]
    yp = frame[:, :, 2:3]
    vxp = frame[:, :, 3:4]
    vyp = frame[:, :, 4:5]
    fx_np, fy_np = _field_xy()
    return _run(xp, yp, vxp, vyp, jnp.asarray(fx_np), jnp.asarray(fy_np))
